# baseline (speedup 1.0000x reference)
import jax
import jax.numpy as jnp
from jax.experimental import pallas as pl
from jax.experimental.pallas import tpu as pltpu

_TAU = 0.07
_TOPK = 128


def _body(feat_ref, score_hbm, hm_ref, pseudo_ref, numer_ref, count_ref,
          scr_ref, dma_sem):
    f = feat_ref[0]
    hm = hm_ref[0]
    c, hw = f.shape
    nc = hm.shape[0]
    b = pl.program_id(0)
    hp = jax.lax.Precision.HIGHEST

    inv = 1.0 / jnp.maximum(jnp.sqrt(jnp.sum(f * f, axis=0, keepdims=True)),
                            1e-12)
    fn_hi = (f * inv).astype(jnp.bfloat16)
    fn_lo = (f * inv - fn_hi.astype(jnp.float32)).astype(jnp.bfloat16)

    n_iota = jax.lax.broadcasted_iota(jnp.int32, (1, hw), 1).astype(jnp.float32)
    li = jnp.max((n_iota + 1.0) * hm, axis=1, keepdims=True) - 1.0
    present = (li >= 0.0).astype(jnp.float32)

    onehot = jnp.logical_and(n_iota == li, li >= 0.0).astype(jnp.bfloat16)
    q = (jax.lax.dot_general(onehot, fn_hi, (((1,), (1,)), ((), ())),
                             preferred_element_type=jnp.float32) +
         jax.lax.dot_general(onehot, fn_lo, (((1,), (1,)), ((), ())),
                             preferred_element_type=jnp.float32))

    hm16 = hm.astype(jnp.bfloat16)
    k0r = (jax.lax.dot_general(hm16, fn_hi, (((1,), (1,)), ((), ())),
                               preferred_element_type=jnp.float32) +
           jax.lax.dot_general(hm16, fn_lo, (((1,), (1,)), ((), ())),
                               preferred_element_type=jnp.float32))
    k0 = k0r / jnp.maximum(
        jnp.sqrt(jnp.sum(k0r * k0r, axis=1, keepdims=True)), 1e-12)

    free = (jnp.sum(hm, axis=0, keepdims=True) == 0.0).astype(jnp.float32)
    nfree = jnp.sum(free)

    L0 = jax.lax.dot_general(q, k0, (((1,), (1,)), ((), ())),
                             precision=hp,
                             preferred_element_type=jnp.float32) / _TAU
    sim_sum0 = jax.lax.dot_general(jnp.exp(L0), present,
                                   (((1,), (0,)), ((), ())),
                                   precision=hp,
                                   preferred_element_type=jnp.float32)
    eye = (jax.lax.broadcasted_iota(jnp.int32, (nc, nc), 0) ==
           jax.lax.broadcasted_iota(jnp.int32, (nc, nc), 1)).astype(jnp.float32)
    diag = jnp.sum(L0 * eye, axis=1, keepdims=True)

    @pl.when(nfree == 0.0)
    def _no_free_positions():
        pseudo_ref[0] = jnp.zeros((nc, hw), jnp.float32)
        ll = diag - jnp.log(sim_sum0)
        numer_ref[0] = jnp.full((1, 128), jnp.sum(ll * present))
        count_ref[0] = jnp.full((1, 128), jnp.sum(present))

    @pl.when(nfree > 0.0)
    def _with_free_positions():
        copy = pltpu.make_async_copy(score_hbm.at[b], scr_ref, dma_sem)
        copy.start()
        copy.wait()
        sc = scr_ref[...]

        ms = sc * free * present
        maxv = jnp.max(ms, axis=0, keepdims=True)
        posF = (maxv > 0.0).astype(jnp.float32)
        npos = jnp.sum(posF)
        lin = jax.lax.broadcasted_iota(jnp.int32, (1, hw), 1)

        def _exact_topk(_):
            def step(i, carry):
                v, s = carry
                m = jnp.max(v)
                j = jnp.min(jnp.where(v == m, lin, hw))
                pick = lin == j
                return jnp.where(pick, -jnp.inf, v), jnp.maximum(
                    s, pick.astype(jnp.float32))
            _, s = jax.lax.fori_loop(
                0, _TOPK, step, (maxv, jnp.zeros((1, hw), jnp.float32)))
            return s

        selF = jax.lax.cond(npos > float(_TOPK), _exact_topk,
                            lambda _: posF, 0)

        k_iota = jax.lax.broadcasted_iota(jnp.int32, (nc, 1), 0)
        cidx = jnp.min(jnp.where(ms == maxv, k_iota, nc), axis=0,
                       keepdims=True)
        P = selF * (cidx == k_iota).astype(jnp.float32)
        pseudo_ref[0] = 0.9 * P

        fn = f * inv
        L = jax.lax.dot_general(q, fn, (((1,), (0,)), ((), ())),
                                precision=hp,
                                preferred_element_type=jnp.float32) / _TAU
        sim_sum_top = jnp.sum(jnp.exp(L) * selF, axis=1, keepdims=True)
        lss = jnp.log(sim_sum_top + sim_sum0)
        cnt = jnp.sum(P, axis=1, keepdims=True)
        pos_logsum = jnp.sum(P * L, axis=1, keepdims=True)
        ll = ((pos_logsum - cnt * lss) + (diag - lss)) / (cnt + 1.0)
        numer_ref[0] = jnp.full((1, 128), jnp.sum(ll * present))
        count_ref[0] = jnp.full((1, 128), jnp.sum(present))


def kernel(feat, score, hm):
    bs, c, h, w = feat.shape
    nc = hm.shape[1]
    hw = h * w
    ff = feat.reshape(bs, c, hw)
    sf = score.reshape(bs, nc, hw)
    hf = hm.reshape(bs, nc, hw)
    pseudo, numer, count = pl.pallas_call(
        _body,
        grid=(bs,),
        in_specs=[
            pl.BlockSpec((1, c, hw), lambda b: (b, 0, 0)),
            pl.BlockSpec(memory_space=pl.ANY),
            pl.BlockSpec((1, nc, hw), lambda b: (b, 0, 0)),
        ],
        out_specs=[
            pl.BlockSpec((1, nc, hw), lambda b: (b, 0, 0)),
            pl.BlockSpec((1, 1, 128), lambda b: (b, 0, 0)),
            pl.BlockSpec((1, 1, 128), lambda b: (b, 0, 0)),
        ],
        out_shape=[
            jax.ShapeDtypeStruct((bs, nc, hw), hm.dtype),
            jax.ShapeDtypeStruct((bs, 1, 128), jnp.float32),
            jax.ShapeDtypeStruct((bs, 1, 128), jnp.float32),
        ],
        scratch_shapes=[
            pltpu.VMEM((nc, hw), jnp.float32),
            pltpu.SemaphoreType.DMA,
        ],
        compiler_params=pltpu.CompilerParams(
            dimension_semantics=("parallel",)),
    )(ff, sf, hf)
    loss = -(jnp.sum(numer[:, 0, 0]) / jnp.sum(count[:, 0, 0]))
    return (loss, pseudo.reshape(bs, nc, h, w))


# speedup vs baseline: 1.0030x; 1.0030x over previous
import jax
import jax.numpy as jnp
from jax.experimental import pallas as pl
from jax.experimental.pallas import tpu as pltpu

_TAU = 0.07
_TOPK = 128


def _body(feat_ref, score_hbm, hm_ref, pseudo_ref, numer_ref, count_ref,
          scr_ref, dma_sem):
    f = feat_ref[0]
    hm = hm_ref[0]
    c, hw = f.shape
    nc = hm.shape[0]
    b = pl.program_id(0)
    hp = jax.lax.Precision.HIGHEST

    inv = 1.0 / jnp.maximum(jnp.sqrt(jnp.sum(f * f, axis=0, keepdims=True)),
                            1e-12)
    fn_hi = (f * inv).astype(jnp.bfloat16)
    fn_lo = (f * inv - fn_hi.astype(jnp.float32)).astype(jnp.bfloat16)

    n_iota = jax.lax.broadcasted_iota(jnp.int32, (1, hw), 1).astype(jnp.float32)
    li = jnp.max((n_iota + 1.0) * hm, axis=1, keepdims=True) - 1.0
    present = (li >= 0.0).astype(jnp.float32)

    onehot = jnp.logical_and(n_iota == li, li >= 0.0).astype(jnp.bfloat16)
    q = (jax.lax.dot_general(onehot, fn_hi, (((1,), (1,)), ((), ())),
                             preferred_element_type=jnp.float32) +
         jax.lax.dot_general(onehot, fn_lo, (((1,), (1,)), ((), ())),
                             preferred_element_type=jnp.float32))

    hm16 = hm.astype(jnp.bfloat16)
    k0r = (jax.lax.dot_general(hm16, fn_hi, (((1,), (1,)), ((), ())),
                               preferred_element_type=jnp.float32) +
           jax.lax.dot_general(hm16, fn_lo, (((1,), (1,)), ((), ())),
                               preferred_element_type=jnp.float32))
    k0 = k0r / jnp.maximum(
        jnp.sqrt(jnp.sum(k0r * k0r, axis=1, keepdims=True)), 1e-12)

    free = (jnp.sum(hm, axis=0, keepdims=True) == 0.0).astype(jnp.float32)
    nfree = jnp.sum(free)

    L0 = jax.lax.dot_general(q, k0, (((1,), (1,)), ((), ())),
                             precision=hp,
                             preferred_element_type=jnp.float32) / _TAU
    sim_sum0 = jax.lax.dot_general(jnp.exp(L0), present,
                                   (((1,), (0,)), ((), ())),
                                   precision=hp,
                                   preferred_element_type=jnp.float32)
    eye = (jax.lax.broadcasted_iota(jnp.int32, (nc, nc), 0) ==
           jax.lax.broadcasted_iota(jnp.int32, (nc, nc), 1)).astype(jnp.float32)
    diag = jnp.sum(L0 * eye, axis=1, keepdims=True)

    @pl.when(nfree == 0.0)
    def _no_free_positions():
        pseudo_ref[0] = jnp.zeros((nc, hw), jnp.float32)
        ll = diag - jnp.log(sim_sum0)
        numer_ref[0] = jnp.full((1, 128), jnp.sum(ll * present))
        count_ref[0] = jnp.full((1, 128), jnp.sum(present))

    @pl.when(nfree > 0.0)
    def _with_free_positions():
        copy = pltpu.make_async_copy(score_hbm.at[b], scr_ref, dma_sem)
        copy.start()
        copy.wait()
        sc = scr_ref[...]

        ms = sc * free * present
        maxv = jnp.max(ms, axis=0, keepdims=True)
        posF = (maxv > 0.0).astype(jnp.float32)
        npos = jnp.sum(posF)
        lin = jax.lax.broadcasted_iota(jnp.int32, (1, hw), 1)

        def _exact_topk(_):
            def step(i, carry):
                v, s = carry
                m = jnp.max(v)
                j = jnp.min(jnp.where(v == m, lin, hw))
                pick = lin == j
                return jnp.where(pick, -jnp.inf, v), jnp.maximum(
                    s, pick.astype(jnp.float32))
            _, s = jax.lax.fori_loop(
                0, _TOPK, step, (maxv, jnp.zeros((1, hw), jnp.float32)))
            return s

        selF = jax.lax.cond(npos > float(_TOPK), _exact_topk,
                            lambda _: posF, 0)

        k_iota = jax.lax.broadcasted_iota(jnp.int32, (nc, 1), 0)
        cidx = jnp.min(jnp.where(ms == maxv, k_iota, nc), axis=0,
                       keepdims=True)
        P = selF * (cidx == k_iota).astype(jnp.float32)
        pseudo_ref[0] = 0.9 * P

        fn = f * inv
        L = jax.lax.dot_general(q, fn, (((1,), (0,)), ((), ())),
                                precision=hp,
                                preferred_element_type=jnp.float32) / _TAU
        sim_sum_top = jnp.sum(jnp.exp(L) * selF, axis=1, keepdims=True)
        lss = jnp.log(sim_sum_top + sim_sum0)
        cnt = jnp.sum(P, axis=1, keepdims=True)
        pos_logsum = jnp.sum(P * L, axis=1, keepdims=True)
        ll = ((pos_logsum - cnt * lss) + (diag - lss)) / (cnt + 1.0)
        numer_ref[0] = jnp.full((1, 128), jnp.sum(ll * present))
        count_ref[0] = jnp.full((1, 128), jnp.sum(present))


def kernel(feat, score, hm):
    bs, c, h, w = feat.shape
    nc = hm.shape[1]
    hw = h * w
    ff = feat.reshape(bs, c, hw)
    sf = score.reshape(bs, nc, hw)
    hf = hm.reshape(bs, nc, hw)
    pseudo, numer, count = pl.pallas_call(
        _body,
        grid=(bs,),
        in_specs=[
            pl.BlockSpec((1, c, hw), lambda b: (b, 0, 0)),
            pl.BlockSpec(memory_space=pl.ANY),
            pl.BlockSpec((1, nc, hw), lambda b: (b, 0, 0)),
        ],
        out_specs=[
            pl.BlockSpec((1, nc, hw), lambda b: (b, 0, 0)),
            pl.BlockSpec((1, 1, 128), lambda b: (b, 0, 0)),
            pl.BlockSpec((1, 1, 128), lambda b: (b, 0, 0)),
        ],
        out_shape=[
            jax.ShapeDtypeStruct((bs, nc, hw), hm.dtype),
            jax.ShapeDtypeStruct((bs, 1, 128), jnp.float32),
            jax.ShapeDtypeStruct((bs, 1, 128), jnp.float32),
        ],
        scratch_shapes=[
            pltpu.VMEM((nc, hw), jnp.float32),
            pltpu.SemaphoreType.DMA,
        ],
    )(ff, sf, hf)
    loss = -(jnp.sum(numer[:, 0, 0]) / jnp.sum(count[:, 0, 0]))
    return (loss, pseudo.reshape(bs, nc, h, w))
